# baseline (speedup 1.0000x reference)
import functools

import jax
import jax.numpy as jnp
from jax import lax
from jax.experimental import pallas as pl
from jax.experimental.pallas import tpu as pltpu
from jax.experimental.pallas import tpu_sc as plsc


@functools.lru_cache(maxsize=None)
def _build_gather(V: int, D: int, N: int):
    info = plsc.get_sparse_core_info()
    NC, NS = info.num_cores, info.num_subcores
    NW = NC * NS
    assert N % NW == 0
    b_per_w = N // NW
    n_chunks = 4
    assert b_per_w % n_chunks == 0
    chunk = b_per_w // n_chunks

    mesh = plsc.VectorSubcoreMesh(core_axis_name="c", subcore_axis_name="s")

    @functools.partial(
        pl.kernel,
        out_type=jax.ShapeDtypeStruct((N, D), jnp.float32),
        mesh=mesh,
        compiler_params=pltpu.CompilerParams(use_tc_tiling_on_sc=False),
        scratch_types=[
            pltpu.VMEM((b_per_w,), jnp.int32),
            pltpu.VMEM((2, chunk, D), jnp.float32),
            pltpu.SemaphoreType.DMA,
            pltpu.SemaphoreType.DMA,
        ],
    )
    def gather_kernel(table_hbm, idx_hbm, out_hbm, idx_v, rows_v, sem0, sem1):
        wid = lax.axis_index("s") * NC + lax.axis_index("c")
        base = wid * b_per_w
        pltpu.sync_copy(idx_hbm.at[pl.ds(base, b_per_w)], idx_v)
        sems = (sem0, sem1)
        copies = []
        for c in range(n_chunks):
            buf = c % 2
            cp = pltpu.async_copy(
                table_hbm.at[idx_v.at[pl.ds(c * chunk, chunk)]],
                rows_v.at[buf],
                sems[buf],
            )
            copies.append(cp)
            if c >= 1:
                copies[c - 1].wait()
                pltpu.sync_copy(
                    rows_v.at[(c - 1) % 2],
                    out_hbm.at[pl.ds(base + (c - 1) * chunk, chunk)],
                )
        copies[-1].wait()
        pltpu.sync_copy(
            rows_v.at[(n_chunks - 1) % 2],
            out_hbm.at[pl.ds(base + (n_chunks - 1) * chunk, chunk)],
        )

    return gather_kernel


def kernel(word_embedding, input_token_ids):
    V, D = word_embedding.shape
    B, H = input_token_ids.shape
    N = B * H
    idx = input_token_ids.reshape(-1).astype(jnp.int32)
    flat = _build_gather(V, D, N)(word_embedding, idx)
    return flat.reshape(B, H, D)


# speedup vs baseline: 1.1602x; 1.1602x over previous
import functools

import jax
import jax.numpy as jnp
from jax import lax
from jax.experimental import pallas as pl
from jax.experimental.pallas import tpu as pltpu
from jax.experimental.pallas import tpu_sc as plsc


@functools.lru_cache(maxsize=None)
def _build_gather(V: int, D: int, B: int, H: int):
    info = plsc.get_sparse_core_info()
    NC, NS, L = info.num_cores, info.num_subcores, info.num_lanes
    NW = NC * NS
    assert D == 32 and L == 16 and B % (NW * 8 * L) == 0
    b_per_w = B // NW
    n_chunks = b_per_w // L
    chunk = L * H
    k_per_w = b_per_w * H

    mesh = plsc.VectorSubcoreMesh(core_axis_name="c", subcore_axis_name="s")

    @functools.partial(
        pl.kernel,
        out_type=jax.ShapeDtypeStruct((H, D // 8, NW, 8, L * 8), jnp.float32),
        mesh=mesh,
        compiler_params=pltpu.CompilerParams(
            use_tc_tiling_on_sc=False, needs_layout_passes=False),
        scratch_types=[
            pltpu.VMEM((k_per_w,), jnp.int32),
            pltpu.VMEM((2, chunk, D), jnp.float32),
            pltpu.VMEM((2, H, D // 8, 8, L), jnp.float32),
            pltpu.SemaphoreType.DMA,
            pltpu.SemaphoreType.DMA,
            pltpu.SemaphoreType.DMA,
            pltpu.SemaphoreType.DMA,
        ],
    )
    def gather_kernel(table_hbm, idx_hbm, out_hbm, idx_v, rows_v, stage_v,
                      gsem0, gsem1, wsem0, wsem1):
        wid = lax.axis_index("s") * NC + lax.axis_index("c")
        pltpu.sync_copy(idx_hbm.at[pl.ds(wid * k_per_w, k_per_w)], idx_v)
        gsems = (gsem0, gsem1)
        wsems = (wsem0, wsem1)
        lane = jnp.arange(L, dtype=jnp.int32)
        row0 = lane * H

        def transpose_chunk(buf):
            def body(h, _):
                row_idx = row0 + h
                for d in range(D):
                    col_idx = jnp.full((L,), d, jnp.int32)
                    vec = plsc.load_gather(rows_v.at[buf], [row_idx, col_idx])
                    stage_v[buf, h, d // 8, d % 8, :] = vec
                return 0

            lax.fori_loop(0, H, body, 0)

        gathers = []
        writes = [None, None]
        for c in range(n_chunks):
            buf = c % 2
            gathers.append(pltpu.async_copy(
                table_hbm.at[idx_v.at[pl.ds(c * chunk, chunk)]],
                rows_v.at[buf],
                gsems[buf],
            ))
            if c >= 1:
                pbuf = (c - 1) % 2
                gathers[c - 1].wait()
                if writes[pbuf] is not None:
                    writes[pbuf].wait()
                transpose_chunk(pbuf)
                writes[pbuf] = pltpu.async_copy(
                    stage_v.at[pbuf],
                    out_hbm.at[:, :, wid, :, pl.ds((c - 1) * L, L)],
                    wsems[pbuf],
                )
        lbuf = (n_chunks - 1) % 2
        gathers[-1].wait()
        if writes[lbuf] is not None:
            writes[lbuf].wait()
        transpose_chunk(lbuf)
        writes[lbuf] = pltpu.async_copy(
            stage_v.at[lbuf],
            out_hbm.at[:, :, wid, :, pl.ds((n_chunks - 1) * L, L)],
            wsems[lbuf],
        )
        for w in writes:
            if w is not None:
                w.wait()

    return gather_kernel


def kernel(word_embedding, input_token_ids):
    V, D = word_embedding.shape
    B, H = input_token_ids.shape
    idx = input_token_ids.astype(jnp.int32).reshape(-1)
    out5 = _build_gather(V, D, B, H)(word_embedding, idx)
    return out5.transpose(2, 4, 0, 1, 3).reshape(B, H, D)


# speedup vs baseline: 1.2624x; 1.0880x over previous
import functools

import jax
import jax.numpy as jnp
from jax import lax
from jax.experimental import pallas as pl
from jax.experimental.pallas import tpu as pltpu
from jax.experimental.pallas import tpu_sc as plsc


@functools.lru_cache(maxsize=None)
def _build_gather(V: int, D: int, B: int, H: int):
    info = plsc.get_sparse_core_info()
    NC, NS, L = info.num_cores, info.num_subcores, info.num_lanes
    NW = NC * NS
    assert D == 32 and L == 16 and H == 50 and B % (NW * 8 * L) == 0
    b_per_w = B // NW
    HH = H // 2
    n_bg = b_per_w // L
    chunk = L * HH

    mesh = plsc.VectorSubcoreMesh(core_axis_name="c", subcore_axis_name="s")

    @functools.partial(
        pl.kernel,
        out_type=jax.ShapeDtypeStruct((H, D // 8, NW, 8, L * 8), jnp.float32),
        mesh=mesh,
        compiler_params=pltpu.CompilerParams(
            use_tc_tiling_on_sc=False, needs_layout_passes=False),
        scratch_types=[
            pltpu.VMEM((b_per_w, H), jnp.int32),
            pltpu.VMEM((2, chunk), jnp.int32),
            pltpu.VMEM((2, chunk), jnp.int32),
            pltpu.VMEM((2, chunk, 4 * D), jnp.float32),
            pltpu.VMEM((HH, D // 8, 8, L), jnp.float32),
            pltpu.SemaphoreType.DMA,
            pltpu.SemaphoreType.DMA,
            pltpu.SemaphoreType.DMA,
        ],
    )
    def gather_kernel(table_hbm, idx_hbm, out_hbm, idx_v, rowid_v, colb_v,
                      rows_v, stage_v, gsem0, gsem1, wsem):
        wid = lax.axis_index("s") * NC + lax.axis_index("c")
        pltpu.sync_copy(idx_hbm.at[pl.ds(wid * b_per_w, b_per_w)], idx_v)
        gsems = (gsem0, gsem1)
        lane = jnp.arange(L, dtype=jnp.int32)

        def prep_chunk(buf, bg, hh):
            def body(h, carry):
                hv = jnp.zeros((L,), jnp.int32) + (hh * HH + h)
                vidx = plsc.load_gather(idx_v, [bg * L + lane, hv])
                rowid_v[buf, pl.ds(h * L, L)] = lax.shift_right_logical(vidx, 2)
                colb_v[buf, pl.ds(h * L, L)] = lax.shift_left(
                    jnp.bitwise_and(vidx, 3), 5)
                return carry

            lax.fori_loop(0, HH, body, 0)

        def transpose_chunk(buf):
            def body(h, carry):
                rowv = jnp.zeros((L,), jnp.int32) + h * L + lane
                cb = colb_v[buf, pl.ds(h * L, L)]
                for d0 in range(0, D, 8):
                    vecs = [
                        plsc.load_gather(rows_v.at[buf], [rowv, cb + d])
                        for d in range(d0, d0 + 8)
                    ]
                    for d, vec in zip(range(d0, d0 + 8), vecs):
                        stage_v[h, d // 8, d % 8, :] = vec
                return carry

            lax.fori_loop(0, HH, body, 0)

        def write_chunk(bg, hh):
            return pltpu.async_copy(
                stage_v,
                out_hbm.at[pl.ds(hh * HH, HH), :, wid, :, pl.ds(bg * L, L)],
                wsem,
            )

        n_chunks = 2 * n_bg
        gathers = []
        prev_write = None
        for c in range(n_chunks):
            bg, hh = c // 2, c % 2
            buf = c % 2
            prep_chunk(buf, bg, hh)
            gathers.append(pltpu.async_copy(
                table_hbm.at[rowid_v.at[buf]], rows_v.at[buf], gsems[buf]))
            if c >= 1:
                pbg, phh = (c - 1) // 2, (c - 1) % 2
                gathers[c - 1].wait()
                if prev_write is not None:
                    prev_write.wait()
                transpose_chunk((c - 1) % 2)
                prev_write = write_chunk(pbg, phh)
        gathers[-1].wait()
        prev_write.wait()
        transpose_chunk((n_chunks - 1) % 2)
        write_chunk(n_bg - 1, 1).wait()

    return gather_kernel


def kernel(word_embedding, input_token_ids):
    V, D = word_embedding.shape
    B, H = input_token_ids.shape
    table4 = word_embedding.reshape(V // 4, 4 * D)
    idx2d = input_token_ids.astype(jnp.int32)
    out5 = _build_gather(V, D, B, H)(table4, idx2d)
    return out5.transpose(2, 4, 0, 1, 3).reshape(B, H, D)
